# baseline (speedup 1.0000x reference)
import functools

import jax
import jax.numpy as jnp
from jax import lax
from jax.experimental import pallas as pl
from jax.experimental.pallas import tpu as pltpu
from jax.experimental.pallas import tpu_sc as plsc

VEC = 64
CTX = 20
NOISE = 26
B = 4096
NC = 2
NS = 16
NW = NC * NS
BPW = B // NW
SUB = 16
NCHUNK = BPW // SUB
LANES = 16
NV = VEC // LANES

RT = 2048


def _mxu_t(blk):
    eye = jnp.eye(VEC, dtype=jnp.float32)
    return lax.dot_general(blk, eye, (((0,), (0,)), ((), ())),
                           preferred_element_type=jnp.float32)


def _pack_words(t):
    lo = lax.bitcast_convert_type(
        t[:, 0:VEC // 2].astype(jnp.bfloat16), jnp.uint16).astype(jnp.uint32)
    hi = lax.bitcast_convert_type(
        t[:, VEC // 2:VEC].astype(jnp.bfloat16), jnp.uint16).astype(jnp.uint32)
    return lax.bitcast_convert_type(lo | (hi << 16), jnp.int32)


def _repack2_body(a1_ref, a2_ref, b1_ref, b2_ref, ap_ref, bp_ref):
    ap_ref[:, 0:VEC // 2] = _pack_words(_mxu_t(a1_ref[...]))
    ap_ref[:, VEC // 2:VEC] = _pack_words(_mxu_t(a2_ref[...]))
    bp_ref[:, 0:VEC // 2] = _pack_words(_mxu_t(b1_ref[...]))
    bp_ref[:, VEC // 2:VEC] = _pack_words(_mxu_t(b2_ref[...]))


def _repack1_body(a1_ref, a2_ref, ap_ref):
    ap_ref[:, 0:VEC // 2] = _pack_words(_mxu_t(a1_ref[...]))
    ap_ref[:, VEC // 2:VEC] = _pack_words(_mxu_t(a2_ref[...]))


def _repack2(a, b):
    n = a.shape[1]
    grid = (n + 2 * RT - 1) // (2 * RT)
    nblk = (n + RT - 1) // RT
    lo = pl.BlockSpec((VEC, RT), lambda j: (0, jnp.minimum(2 * j, nblk - 1)))
    hi = pl.BlockSpec(
        (VEC, RT), lambda j: (0, jnp.minimum(2 * j + 1, nblk - 1)))
    return pl.pallas_call(
        _repack2_body,
        grid=(grid,),
        in_specs=[lo, hi, lo, hi],
        out_specs=[pl.BlockSpec((RT, VEC), lambda j: (j, 0)),
                   pl.BlockSpec((RT, VEC), lambda j: (j, 0))],
        out_shape=[jax.ShapeDtypeStruct((grid * RT, VEC), jnp.int32)] * 2,
    )(a, a, b, b)


def _repack1(a):
    n = a.shape[1]
    grid = (n + 2 * RT - 1) // (2 * RT)
    nblk = (n + RT - 1) // RT
    lo = pl.BlockSpec((VEC, RT), lambda j: (0, jnp.minimum(2 * j, nblk - 1)))
    hi = pl.BlockSpec(
        (VEC, RT), lambda j: (0, jnp.minimum(2 * j + 1, nblk - 1)))
    return pl.pallas_call(
        _repack1_body,
        grid=(grid,),
        in_specs=[lo, hi],
        out_specs=pl.BlockSpec((RT, VEC), lambda j: (j, 0)),
        out_shape=jax.ShapeDtypeStruct((grid * RT, VEC), jnp.int32),
    )(a, a)


def _sc_forward(ctxp_hbm, docp_hbm, noisep_hbm, offs_hbm,
                wp_hbm, dp_hbm, op_hbm, out_hbm,
                ctxp_v, docp_v, noisep_v, offs_v,
                w_rows, d_rows, og_rows, out_v, sem):
    wid = lax.axis_index("s") * NC + lax.axis_index("c")
    lane = lax.iota(jnp.int32, LANES)
    perms = [lane ^ (1 << k) for k in range(4)]
    for chunk in range(NCHUNK):
        cb = wid * BPW + chunk * SUB
        pltpu.sync_copy(ctxp_hbm.at[pl.ds(cb * CTX, SUB * CTX)], ctxp_v)
        pltpu.sync_copy(docp_hbm.at[pl.ds(cb, SUB)], docp_v)
        pltpu.sync_copy(noisep_hbm.at[pl.ds(cb * NOISE, SUB * NOISE)],
                        noisep_v)
        pltpu.sync_copy(offs_hbm.at[pl.ds(cb, SUB)], offs_v)
        pltpu.async_copy(wp_hbm.at[ctxp_v], w_rows, sem).wait()
        pltpu.async_copy(dp_hbm.at[docp_v], d_rows, sem).wait()
        pltpu.async_copy(op_hbm.at[noisep_v], og_rows, sem).wait()

        def body(b, carry):
            oa = offs_v[b, pl.ds(0, LANES)]
            ob = offs_v[b, pl.ds(LANES, LANES)]
            oc = offs_v[b, pl.ds(2 * LANES, LANES)]

            def off_at(j):
                src, idx = (oa, j) if j < LANES else (
                    (ob, j - LANES) if j < 2 * LANES else (oc, j - 2 * LANES))
                return src[idx]

            mhi = jnp.full((LANES,), -65536, jnp.int32)

            def row_f32(rows, r, off):
                out = []
                for u in range(2):
                    w32 = rows[r, pl.ds(off + u * LANES, LANES)]
                    a = lax.bitcast_convert_type(w32 << 16, jnp.float32)
                    bb = lax.bitcast_convert_type(w32 & mhi, jnp.float32)
                    out += [a, bb]
                return out

            doff = off_at(0)
            xs = row_f32(d_rows, b, doff)
            for c in range(CTX):
                r = b * CTX + c
                ws = row_f32(w_rows, r, off_at(c + 1))
                for v in range(NV):
                    xs[v] = xs[v] + ws[v]
            o_lo = jnp.zeros((LANES,), jnp.float32)
            o_hi = jnp.zeros((LANES,), jnp.float32)
            for n in range(NOISE):
                r = b * NOISE + n
                os_ = row_f32(og_rows, r, off_at(n + 1 + CTX))
                p = jnp.zeros((LANES,), jnp.float32)
                for v in range(NV):
                    p = p + xs[v] * os_[v]
                for perm in perms:
                    p = p + jnp.take(p, perm)
                if n < LANES:
                    o_lo = jnp.where(lane == n, p, o_lo)
                if n >= NOISE - LANES:
                    o_hi = jnp.where(lane == n - (NOISE - LANES), p, o_hi)
            out_v[b, pl.ds(0, LANES)] = o_lo
            out_v[b, pl.ds(NOISE - LANES, LANES)] = o_hi
            return carry

        lax.fori_loop(0, SUB, body, 0)
        pltpu.sync_copy(out_v, out_hbm.at[pl.ds(cb, SUB)])


@functools.lru_cache(maxsize=1)
def _sc_call():
    return pl.kernel(
        _sc_forward,
        out_type=jax.ShapeDtypeStruct((B, NOISE), jnp.float32),
        compiler_params=pltpu.CompilerParams(use_tc_tiling_on_sc=False),
        mesh=plsc.VectorSubcoreMesh(
            core_axis_name="c", subcore_axis_name="s", num_cores=NC,
            num_subcores=NS,
        ),
        scratch_types=[
            pltpu.VMEM((SUB * CTX,), jnp.int32),
            pltpu.VMEM((SUB,), jnp.int32),
            pltpu.VMEM((SUB * NOISE,), jnp.int32),
            pltpu.VMEM((SUB, VEC), jnp.int32),
            pltpu.VMEM((SUB * CTX, VEC), jnp.int32),
            pltpu.VMEM((SUB, VEC), jnp.int32),
            pltpu.VMEM((SUB * NOISE, VEC), jnp.int32),
            pltpu.VMEM((SUB, NOISE), jnp.float32),
            pltpu.SemaphoreType.DMA,
        ],
    )


_RTL = RT.bit_length() - 1


def _pair_ids(ids):
    pair = (ids >> (_RTL + 1)) * RT + (ids & (RT - 1))
    off = ((ids >> _RTL) & 1) * (VEC // 2)
    return pair, off


def kernel(context_ids, doc_ids, target_noise_ids, D, W, O):
    wp, op_ = _repack2(W.T, O)
    dp = _repack1(D.T)
    ctx_flat = context_ids.reshape(-1).astype(jnp.int32)
    noise_flat = target_noise_ids.reshape(-1).astype(jnp.int32)
    doc = doc_ids.astype(jnp.int32)
    cp, co = _pair_ids(ctx_flat)
    dcp, dco = _pair_ids(doc)
    np_, no = _pair_ids(noise_flat)
    offs = jnp.concatenate(
        [dco[:, None], co.reshape(B, CTX), no.reshape(B, NOISE),
         jnp.zeros((B, VEC - 1 - CTX - NOISE), jnp.int32)], axis=1)
    return _sc_call()(cp, dcp, np_, offs, wp, dp, op_)


# speedup vs baseline: 1.7160x; 1.7160x over previous
import functools

import jax
import jax.numpy as jnp
from jax import lax
from jax.experimental import pallas as pl
from jax.experimental.pallas import tpu as pltpu
from jax.experimental.pallas import tpu_sc as plsc

VEC = 64
CTX = 20
NOISE = 26
B = 4096
NC = 2
NS = 16
NW = NC * NS
BPW = B // NW
SUB = 16
NCHUNK = BPW // SUB
LANES = 16
NV = VEC // LANES

RT = 2048


def _mxu_t(blk):
    eye = jnp.eye(VEC, dtype=jnp.float32)
    return lax.dot_general(blk, eye, (((0,), (0,)), ((), ())),
                           preferred_element_type=jnp.float32)


def _repack2_body(a1_ref, a2_ref, b1_ref, b2_ref, ap_ref, bp_ref):
    ap_ref[:, 0:VEC] = _mxu_t(a1_ref[...])
    ap_ref[:, VEC:2 * VEC] = _mxu_t(a2_ref[...])
    bp_ref[:, 0:VEC] = _mxu_t(b1_ref[...])
    bp_ref[:, VEC:2 * VEC] = _mxu_t(b2_ref[...])


def _repack1_body(a1_ref, a2_ref, ap_ref):
    ap_ref[:, 0:VEC] = _mxu_t(a1_ref[...])
    ap_ref[:, VEC:2 * VEC] = _mxu_t(a2_ref[...])


def _repack2(a, b):
    n = a.shape[1]
    grid = (n + 2 * RT - 1) // (2 * RT)
    nblk = (n + RT - 1) // RT
    lo = pl.BlockSpec((VEC, RT), lambda j: (0, jnp.minimum(2 * j, nblk - 1)))
    hi = pl.BlockSpec(
        (VEC, RT), lambda j: (0, jnp.minimum(2 * j + 1, nblk - 1)))
    return pl.pallas_call(
        _repack2_body,
        grid=(grid,),
        in_specs=[lo, hi, lo, hi],
        out_specs=[pl.BlockSpec((RT, 2 * VEC), lambda j: (j, 0)),
                   pl.BlockSpec((RT, 2 * VEC), lambda j: (j, 0))],
        out_shape=[jax.ShapeDtypeStruct((grid * RT, 2 * VEC),
                                        jnp.float32)] * 2,
    )(a, a, b, b)


def _repack1(a):
    n = a.shape[1]
    grid = (n + 2 * RT - 1) // (2 * RT)
    nblk = (n + RT - 1) // RT
    lo = pl.BlockSpec((VEC, RT), lambda j: (0, jnp.minimum(2 * j, nblk - 1)))
    hi = pl.BlockSpec(
        (VEC, RT), lambda j: (0, jnp.minimum(2 * j + 1, nblk - 1)))
    return pl.pallas_call(
        _repack1_body,
        grid=(grid,),
        in_specs=[lo, hi],
        out_specs=pl.BlockSpec((RT, 2 * VEC), lambda j: (j, 0)),
        out_shape=jax.ShapeDtypeStruct((grid * RT, 2 * VEC), jnp.float32),
    )(a, a)


def _sc_forward(ctxp_hbm, docp_hbm, noisep_hbm, offs_hbm,
                wp_hbm, dp_hbm, op_hbm, out_hbm,
                ctxp_v, docp_v, noisep_v, offs_v,
                w_rows, d_rows, og_rows, out_v, sem):
    wid = lax.axis_index("s") * NC + lax.axis_index("c")
    lane = lax.iota(jnp.int32, LANES)
    perms = [lane ^ (1 << k) for k in range(4)]
    for chunk in range(NCHUNK):
        cb = wid * BPW + chunk * SUB
        pltpu.sync_copy(ctxp_hbm.at[pl.ds(cb * CTX, SUB * CTX)], ctxp_v)
        pltpu.sync_copy(docp_hbm.at[pl.ds(cb, SUB)], docp_v)
        pltpu.sync_copy(noisep_hbm.at[pl.ds(cb * NOISE, SUB * NOISE)],
                        noisep_v)
        pltpu.sync_copy(offs_hbm.at[pl.ds(cb, SUB)], offs_v)
        pltpu.async_copy(wp_hbm.at[ctxp_v], w_rows, sem).wait()
        pltpu.async_copy(dp_hbm.at[docp_v], d_rows, sem).wait()
        pltpu.async_copy(op_hbm.at[noisep_v], og_rows, sem).wait()

        def body(b, carry):
            oa = offs_v[b, pl.ds(0, LANES)]
            ob = offs_v[b, pl.ds(LANES, LANES)]
            oc = offs_v[b, pl.ds(2 * LANES, LANES)]

            def off_at(j):
                src, idx = (oa, j) if j < LANES else (
                    (ob, j - LANES) if j < 2 * LANES else (oc, j - 2 * LANES))
                return src[idx]

            doff = off_at(0)
            xs = [d_rows[b, pl.ds(doff + v * LANES, LANES)]
                  for v in range(NV)]
            for c in range(CTX):
                r = b * CTX + c
                off = off_at(c + 1)
                for v in range(NV):
                    xs[v] = xs[v] + w_rows[r, pl.ds(off + v * LANES, LANES)]
            o_lo = jnp.zeros((LANES,), jnp.float32)
            o_hi = jnp.zeros((LANES,), jnp.float32)
            for n in range(NOISE):
                r = b * NOISE + n
                noff = off_at(n + 1 + CTX)
                p = jnp.zeros((LANES,), jnp.float32)
                for v in range(NV):
                    p = p + xs[v] * og_rows[r, pl.ds(noff + v * LANES, LANES)]
                for perm in perms:
                    p = p + jnp.take(p, perm)
                if n < LANES:
                    o_lo = jnp.where(lane == n, p, o_lo)
                if n >= NOISE - LANES:
                    o_hi = jnp.where(lane == n - (NOISE - LANES), p, o_hi)
            out_v[b, pl.ds(0, LANES)] = o_lo
            out_v[b, pl.ds(NOISE - LANES, LANES)] = o_hi
            return carry

        lax.fori_loop(0, SUB, body, 0)
        pltpu.sync_copy(out_v, out_hbm.at[pl.ds(cb, SUB)])


@functools.lru_cache(maxsize=1)
def _sc_call():
    return pl.kernel(
        _sc_forward,
        out_type=jax.ShapeDtypeStruct((B, NOISE), jnp.float32),
        compiler_params=pltpu.CompilerParams(use_tc_tiling_on_sc=False),
        mesh=plsc.VectorSubcoreMesh(
            core_axis_name="c", subcore_axis_name="s", num_cores=NC,
            num_subcores=NS,
        ),
        scratch_types=[
            pltpu.VMEM((SUB * CTX,), jnp.int32),
            pltpu.VMEM((SUB,), jnp.int32),
            pltpu.VMEM((SUB * NOISE,), jnp.int32),
            pltpu.VMEM((SUB, VEC), jnp.int32),
            pltpu.VMEM((SUB * CTX, 2 * VEC), jnp.float32),
            pltpu.VMEM((SUB, 2 * VEC), jnp.float32),
            pltpu.VMEM((SUB * NOISE, 2 * VEC), jnp.float32),
            pltpu.VMEM((SUB, NOISE), jnp.float32),
            pltpu.SemaphoreType.DMA,
        ],
    )


_RTL = RT.bit_length() - 1


def _pair_ids(ids):
    pair = (ids >> (_RTL + 1)) * RT + (ids & (RT - 1))
    off = ((ids >> _RTL) & 1) * VEC
    return pair, off


def kernel(context_ids, doc_ids, target_noise_ids, D, W, O):
    wp, op_ = _repack2(W.T, O)
    dp = _repack1(D.T)
    ctx_flat = context_ids.reshape(-1).astype(jnp.int32)
    noise_flat = target_noise_ids.reshape(-1).astype(jnp.int32)
    doc = doc_ids.astype(jnp.int32)
    cp, co = _pair_ids(ctx_flat)
    dcp, dco = _pair_ids(doc)
    np_, no = _pair_ids(noise_flat)
    offs = jnp.concatenate(
        [dco[:, None], co.reshape(B, CTX), no.reshape(B, NOISE),
         jnp.zeros((B, VEC - 1 - CTX - NOISE), jnp.int32)], axis=1)
    return _sc_call()(cp, dcp, np_, offs, wp, dp, op_)
